# Initial kernel scaffold; baseline (speedup 1.0000x reference)
import jax
import jax.numpy as jnp
from jax.experimental import pallas as pl


def kernel(x):
    raise NotImplementedError("write your pallas kernel here")


# speedup vs baseline: 1.0163x; 1.0163x over previous
import jax
import jax.numpy as jnp
from jax import lax
from jax.experimental import pallas as pl
from jax.experimental.pallas import tpu as pltpu
from jax.experimental.pallas import tpu_sc as plsc

R = 128
C = 32768
L = 16
NC = 2
NS = 16
NW = NC * NS
RPW = R // NW
NCHUNK = C // L

NEG_INF = float("-inf")
IMAX = 2**31 - 1

_GATHER_DNUMS = lax.GatherDimensionNumbers(
    offset_dims=(), collapsed_slice_dims=(0,), start_index_map=(0,))


def _dyn_gather(x, idx):
    return lax.gather(
        x, idx.reshape(L, 1), dimension_numbers=_GATHER_DNUMS,
        slice_sizes=(1,), mode=lax.GatherScatterMode.PROMISE_IN_BOUNDS)


def _topk_body(x_hbm, vals_hbm, idx_hbm, xbuf, vout, iout, sem0, sem1):
    cid = lax.axis_index("c")
    sid = lax.axis_index("s")
    wid = sid * NC + cid
    base_row = wid * RPW

    sems = (sem0, sem1)
    pending = [None, None]
    pending[0] = pltpu.async_copy(x_hbm.at[base_row], xbuf.at[0], sems[0])
    lane = lax.iota(jnp.int32, L)

    for r in range(RPW):
        buf = r % 2
        if r + 1 < RPW:
            pending[1 - buf] = pltpu.async_copy(
                x_hbm.at[base_row + r + 1], xbuf.at[1 - buf], sems[1 - buf])
        pending[buf].wait()

        def body(i, carry):
            m1, m2, m3, i1, i2, i3, ivec = carry
            v = xbuf[buf, pl.ds(i * L, L)]
            gt1 = v > m1
            gt2 = v > m2
            gt3 = v > m3
            nm1 = jnp.maximum(v, m1)
            nm2 = jnp.where(gt1, m1, jnp.where(gt2, v, m2))
            nm3 = jnp.where(gt2, m2, jnp.where(gt3, v, m3))
            ni1 = jnp.where(gt1, ivec, i1)
            ni2 = jnp.where(gt1, i1, jnp.where(gt2, ivec, i2))
            ni3 = jnp.where(gt2, i2, jnp.where(gt3, ivec, i3))
            return (nm1, nm2, nm3, ni1, ni2, ni3, ivec + L)

        neg = jnp.full((L,), NEG_INF, jnp.float32)
        zero_i = jnp.zeros((L,), jnp.int32)
        m1, m2, m3, i1, i2, i3, _ = lax.fori_loop(
            0, NCHUNK, body, (neg, neg, neg, zero_i, zero_i, zero_i, lane),
            unroll=8)

        rv = jnp.zeros((L,), jnp.float32)
        ri = jnp.zeros((L,), jnp.int32)
        for k in range(3):
            vmax, imin = m1, i1
            for s in (8, 4, 2, 1):
                perm = lane ^ s
                ov = _dyn_gather(vmax, perm)
                oi = _dyn_gather(imin, perm)
                take = (ov > vmax) | ((ov == vmax) & (oi < imin))
                vmax = jnp.where(take, ov, vmax)
                imin = jnp.where(take, oi, imin)
            win = (m1 == vmax) & (i1 == imin)
            rv = jnp.where(lane == k, vmax, rv)
            ri = jnp.where(lane == k, imin, ri)
            m1 = jnp.where(win, m2, m1)
            m2 = jnp.where(win, m3, m2)
            m3 = jnp.where(win, NEG_INF, m3)
            i1 = jnp.where(win, i2, i1)
            i2 = jnp.where(win, i3, i2)

        vout[r, :] = rv
        iout[r, :] = ri

    pltpu.sync_copy(vout, vals_hbm.at[pl.ds(base_row, RPW)])
    pltpu.sync_copy(iout, idx_hbm.at[pl.ds(base_row, RPW)])


@jax.jit
def kernel(x):
    mesh = plsc.VectorSubcoreMesh(core_axis_name="c", subcore_axis_name="s")
    f = pl.kernel(
        _topk_body,
        out_type=[jax.ShapeDtypeStruct((R, L), jnp.float32),
                  jax.ShapeDtypeStruct((R, L), jnp.int32)],
        mesh=mesh,
        scratch_types=[
            pltpu.VMEM((2, C), jnp.float32),
            pltpu.VMEM((RPW, L), jnp.float32),
            pltpu.VMEM((RPW, L), jnp.int32),
            pltpu.SemaphoreType.DMA,
            pltpu.SemaphoreType.DMA,
        ],
    )
    vals, idx = f(x)
    return vals[:, :3], idx[:, :3]
